# baseline (speedup 1.0000x reference)
import functools

import jax
import jax.numpy as jnp
from jax import lax
from jax.experimental import pallas as pl
from jax.experimental.pallas import tpu as pltpu
from jax.experimental.pallas import tpu_sc as plsc

B, H, W, C = 4, 112, 112, 192
HW = H * W
P = (2 * H) * (2 * W)
CB = 16
NCB = C // CB
NREG = B * NCB
NC, NS, L = 2, 16, 16
IN_T = HW * CB // NS
CH = IN_T // 128
ACC = P * CB
OUT_T = ACC // NS
ZN = IN_T

_mesh = plsc.VectorSubcoreMesh(core_axis_name="c", subcore_axis_name="s")


@functools.partial(
    pl.kernel,
    out_type=jax.ShapeDtypeStruct((NREG, P * CB), jnp.float32),
    mesh=_mesh,
    compiler_params=pltpu.CompilerParams(use_tc_tiling_on_sc=False),
    scratch_types=[
        pltpu.VMEM_SHARED((ACC,), jnp.float32),
        pltpu.VMEM((IN_T,), jnp.int32),
        pltpu.VMEM((IN_T,), jnp.float32),
        pltpu.VMEM((CH, 128), jnp.int32),
        pltpu.VMEM((CH, 128), jnp.float32),
        pltpu.VMEM((ZN,), jnp.float32),
    ],
)
def _unpool(upd_hbm, msk_hbm, out_hbm, acc, msk_w, upd_w, idx2d, val2d,
            zeros):
    core = lax.axis_index("c")
    sub = lax.axis_index("s")
    in0 = sub * IN_T
    out0 = sub * OUT_T
    iota = lax.broadcasted_iota(jnp.int32, (L,), 0)
    z16 = jnp.zeros((L,), jnp.float32)
    third = jnp.float32(1.0 / 3.0)

    @pl.loop(0, ZN // L)
    def _(g):
        zeros[pl.ds(g * L, L)] = z16

    @pl.loop(0, OUT_T // ZN)
    def _(q):
        pltpu.sync_copy(zeros, acc.at[pl.ds(out0 + q * ZN, ZN)])

    plsc.subcore_barrier()

    @pl.loop(0, B)
    def _(b):
        @pl.loop(0, NCB // NC)
        def _(cbs):
            reg = b * NCB + cbs * NC + core
            pltpu.sync_copy(msk_hbm.at[reg, pl.ds(in0, IN_T)], msk_w)
            pltpu.sync_copy(upd_hbm.at[reg, pl.ds(in0, IN_T)], upd_w)

            @pl.loop(0, CH)
            def _(j):
                for jj in range(8):
                    o = jj * L
                    vm = msk_w[pl.ds(j * 128 + o, L)]
                    u = lax.shift_right_logical(vm, 6)
                    p = (u.astype(jnp.float32) * third).astype(jnp.int32)
                    idx2d[j, pl.ds(o, L)] = p * CB + iota
                    val2d[j, pl.ds(o, L)] = upd_w[pl.ds(j * 128 + o, L)]

            @pl.loop(0, CH)
            def _(j):
                pltpu.sync_copy(val2d.at[j], acc.at[idx2d.at[j]], add=True)

            plsc.subcore_barrier()
            pltpu.sync_copy(acc.at[pl.ds(out0, OUT_T)],
                            out_hbm.at[reg, pl.ds(out0, OUT_T)])

            @pl.loop(0, OUT_T // ZN)
            def _(q):
                pltpu.sync_copy(zeros, acc.at[pl.ds(out0 + q * ZN, ZN)])

            plsc.subcore_barrier()


def kernel(updates, mask):
    upd5 = updates.reshape(B, HW, NCB, CB).transpose(0, 2, 1, 3)
    msk5 = mask.astype(jnp.int32).reshape(B, HW, NCB, CB).transpose(0, 2, 1, 3)
    out = _unpool(upd5.reshape(NREG, HW * CB), msk5.reshape(NREG, HW * CB))
    return (out.reshape(B, NCB, P, CB).transpose(0, 2, 1, 3)
            .reshape(B, 2 * H, 2 * W, C))


# speedup vs baseline: 1.3419x; 1.3419x over previous
import functools

import jax
import jax.numpy as jnp
from jax import lax
from jax.experimental import pallas as pl
from jax.experimental.pallas import tpu as pltpu
from jax.experimental.pallas import tpu_sc as plsc

B, H, W, C = 4, 112, 112, 192
HW = H * W
P = (2 * H) * (2 * W)
CB = 16
NCB = C // CB
NREG = B * NCB
NC, NS, L = 2, 16, 16
ROWS = HW // NS
IN_T = ROWS * CB
CH = IN_T // 128
ACC = P * CB
OUT_T = ACC // NS
ZN = 3136

_mesh = plsc.VectorSubcoreMesh(core_axis_name="c", subcore_axis_name="s")


@functools.partial(
    pl.kernel,
    out_type=jax.ShapeDtypeStruct((NREG, P * CB), jnp.float32),
    mesh=_mesh,
    compiler_params=pltpu.CompilerParams(use_tc_tiling_on_sc=False),
    scratch_types=[
        pltpu.VMEM_SHARED((ACC,), jnp.float32),
        pltpu.VMEM((ROWS, CB), jnp.int32),
        pltpu.VMEM((ROWS, CB), jnp.int32),
        pltpu.VMEM((ROWS, CB), jnp.float32),
        pltpu.VMEM((ROWS, CB), jnp.float32),
        pltpu.VMEM((CH, 128), jnp.int32),
        pltpu.VMEM((CH, 128), jnp.float32),
        pltpu.VMEM((ZN,), jnp.float32),
        pltpu.SemaphoreType.DMA,
        pltpu.SemaphoreType.DMA,
    ],
)
def _unpool(upd_hbm, msk_hbm, out_hbm, acc, msk0, msk1, upd0, upd1,
            idx2d, val2d, zeros, sem_in, sem_sc):
    core = lax.axis_index("c")
    sub = lax.axis_index("s")
    row0 = sub * ROWS
    out0 = sub * OUT_T
    iota = lax.broadcasted_iota(jnp.int32, (L,), 0)
    z16 = jnp.zeros((L,), jnp.float32)
    third = jnp.float32(1.0 / 3.0)
    msk_w = (msk0, msk1)
    upd_w = (upd0, upd1)

    @pl.loop(0, ZN // L)
    def _(g):
        zeros[pl.ds(g * L, L)] = z16

    @pl.loop(0, OUT_T // ZN)
    def _(q):
        pltpu.sync_copy(zeros, acc.at[pl.ds(out0 + q * ZN, ZN)])

    def fire_inputs(b, cbs, slot):
        c0 = cbs * (CB * NC) + core * CB
        src_m = msk_hbm.at[b, pl.ds(row0, ROWS), pl.ds(c0, CB)]
        src_u = upd_hbm.at[b, pl.ds(row0, ROWS), pl.ds(c0, CB)]
        pltpu.async_copy(src_m, msk_w[slot], sem_in)
        pltpu.async_copy(src_u, upd_w[slot], sem_in)
        return src_m, src_u

    def drain_inputs(b, cbs, slot):
        c0 = cbs * (CB * NC) + core * CB
        src_m = msk_hbm.at[b, pl.ds(row0, ROWS), pl.ds(c0, CB)]
        src_u = upd_hbm.at[b, pl.ds(row0, ROWS), pl.ds(c0, CB)]
        pltpu.make_async_copy(src_m, msk_w[slot], sem_in).wait()
        pltpu.make_async_copy(src_u, upd_w[slot], sem_in).wait()

    fire_inputs(jnp.int32(0), jnp.int32(0), 0)
    plsc.subcore_barrier()

    @pl.loop(0, B)
    def _(b):
        for cbs in range(NCB // NC):
            slot = cbs % 2
            reg = b * NCB + cbs * NC + core
            drain_inputs(b, cbs, slot)
            mw, uw = msk_w[slot], upd_w[slot]

            @pl.loop(0, CH)
            def _(j):
                for jj in range(8):
                    o = jj * L
                    row = j * 8 + jj
                    vm = mw[row, pl.ds(0, L)]
                    u = lax.shift_right_logical(vm, 6)
                    p = (u.astype(jnp.float32) * third).astype(jnp.int32)
                    idx2d[j, pl.ds(o, L)] = p * CB + iota
                    val2d[j, pl.ds(o, L)] = uw[row, pl.ds(0, L)]

            @pl.loop(0, CH)
            def _(j):
                pltpu.async_copy(val2d.at[j], acc.at[idx2d.at[j]], sem_sc,
                                 add=True)

            @pl.loop(0, CH)
            def _(j):
                pltpu.make_async_copy(val2d.at[j], acc.at[idx2d.at[j]],
                                      sem_sc).wait()

            if cbs + 1 < NCB // NC:
                fire_inputs(b, jnp.int32(cbs + 1), (cbs + 1) % 2)
            else:
                fire_inputs(jnp.minimum(b + 1, B - 1), jnp.int32(0), 0)

            plsc.subcore_barrier()
            pltpu.sync_copy(acc.at[pl.ds(out0, OUT_T)],
                            out_hbm.at[reg, pl.ds(out0, OUT_T)])

            @pl.loop(0, OUT_T // ZN)
            def _(q):
                pltpu.sync_copy(zeros, acc.at[pl.ds(out0 + q * ZN, ZN)])

            plsc.subcore_barrier()

    drain_inputs(jnp.int32(B - 1), 0, 0)


def kernel(updates, mask):
    upd3 = updates.reshape(B, HW, C)
    msk3 = mask.astype(jnp.int32).reshape(B, HW, C)
    out = _unpool(upd3, msk3)
    return (out.reshape(B, NCB, P, CB).transpose(0, 2, 1, 3)
            .reshape(B, 2 * H, 2 * W, C))


# speedup vs baseline: 1.3464x; 1.0034x over previous
import functools

import jax
import jax.numpy as jnp
from jax import lax
from jax.experimental import pallas as pl
from jax.experimental.pallas import tpu as pltpu
from jax.experimental.pallas import tpu_sc as plsc

B, H, W, C = 4, 112, 112, 192
HW = H * W
P = (2 * H) * (2 * W)
CB = 16
NCB = C // CB
NC, NS, L = 2, 16, 16
ROWS = HW // NS
IN_T = ROWS * CB
CH = IN_T // 128
ACC = P * CB
OUT_T = ACC // NS
POS_T = P // NS
RB_POS = POS_T // 4
RB_N = RB_POS * CB
ZN = 3136

_mesh = plsc.VectorSubcoreMesh(core_axis_name="c", subcore_axis_name="s")


@functools.partial(
    pl.kernel,
    out_type=jax.ShapeDtypeStruct((B, P, C), jnp.float32),
    mesh=_mesh,
    compiler_params=pltpu.CompilerParams(use_tc_tiling_on_sc=False),
    scratch_types=[
        pltpu.VMEM_SHARED((ACC,), jnp.float32),
        pltpu.VMEM((ROWS, CB), jnp.int32),
        pltpu.VMEM((ROWS, CB), jnp.float32),
        pltpu.VMEM((CH, 128), jnp.int32),
        pltpu.VMEM((CH, 128), jnp.float32),
        pltpu.VMEM((RB_N,), jnp.float32),
        pltpu.VMEM((RB_POS, CB), jnp.float32),
        pltpu.VMEM((ZN,), jnp.float32),
        pltpu.SemaphoreType.DMA,
        pltpu.SemaphoreType.DMA,
    ],
)
def _unpool(upd_hbm, msk_hbm, out_hbm, acc, msk_w, upd_w, idx2d, val2d,
            rb1d, rb2d, zeros, sem_in, sem_sc):
    core = lax.axis_index("c")
    sub = lax.axis_index("s")
    row0 = sub * ROWS
    out0 = sub * OUT_T
    pos0 = sub * POS_T
    iota = lax.broadcasted_iota(jnp.int32, (L,), 0)
    z16 = jnp.zeros((L,), jnp.float32)
    third = jnp.float32(1.0 / 3.0)

    @pl.loop(0, ZN // L)
    def _(g):
        zeros[pl.ds(g * L, L)] = z16

    @pl.loop(0, OUT_T // ZN)
    def _(q):
        pltpu.sync_copy(zeros, acc.at[pl.ds(out0 + q * ZN, ZN)])

    def in_refs(b, cbs):
        c0 = cbs * (CB * NC) + core * CB
        src_m = msk_hbm.at[b, pl.ds(row0, ROWS), pl.ds(c0, CB)]
        src_u = upd_hbm.at[b, pl.ds(row0, ROWS), pl.ds(c0, CB)]
        return src_m, src_u

    def fire_inputs(b, cbs):
        src_m, src_u = in_refs(b, cbs)
        pltpu.async_copy(src_m, msk_w, sem_in)
        pltpu.async_copy(src_u, upd_w, sem_in)

    def drain_inputs(b, cbs):
        src_m, src_u = in_refs(b, cbs)
        pltpu.make_async_copy(src_m, msk_w, sem_in).wait()
        pltpu.make_async_copy(src_u, upd_w, sem_in).wait()

    fire_inputs(jnp.int32(0), jnp.int32(0))
    plsc.subcore_barrier()

    @pl.loop(0, B)
    def _(b):
        for cbs in range(NCB // NC):
            c0 = cbs * (CB * NC) + core * CB
            drain_inputs(b, cbs)

            @pl.loop(0, CH)
            def _(j):
                for jj in range(8):
                    o = jj * L
                    row = j * 8 + jj
                    vm = msk_w[row, pl.ds(0, L)]
                    u = lax.shift_right_logical(vm, 6)
                    p = (u.astype(jnp.float32) * third).astype(jnp.int32)
                    idx2d[j, pl.ds(o, L)] = p * CB + iota
                    val2d[j, pl.ds(o, L)] = upd_w[row, pl.ds(0, L)]

            if cbs + 1 < NCB // NC:
                fire_inputs(b, jnp.int32(cbs + 1))
            else:
                fire_inputs(jnp.minimum(b + 1, B - 1), jnp.int32(0))

            @pl.loop(0, CH)
            def _(j):
                pltpu.async_copy(val2d.at[j], acc.at[idx2d.at[j]], sem_sc,
                                 add=True)

            @pl.loop(0, CH)
            def _(j):
                pltpu.make_async_copy(val2d.at[j], acc.at[idx2d.at[j]],
                                      sem_sc).wait()

            plsc.subcore_barrier()

            @pl.loop(0, POS_T // RB_POS)
            def _(q):
                off = out0 + q * RB_N
                pltpu.sync_copy(acc.at[pl.ds(off, RB_N)], rb1d)

                @pl.loop(0, RB_POS)
                def _(t):
                    rb2d[t, pl.ds(0, L)] = rb1d[pl.ds(t * L, L)]

                pltpu.sync_copy(
                    rb2d,
                    out_hbm.at[b, pl.ds(pos0 + q * RB_POS, RB_POS),
                               pl.ds(c0, CB)])

                @pl.loop(0, RB_N // ZN)
                def _(z):
                    pltpu.sync_copy(zeros,
                                    acc.at[pl.ds(off + z * ZN, ZN)])

            plsc.subcore_barrier()

    drain_inputs(jnp.int32(B - 1), 0)


def kernel(updates, mask):
    upd3 = updates.reshape(B, HW, C)
    msk3 = mask.astype(jnp.int32).reshape(B, HW, C)
    return _unpool(upd3, msk3).reshape(B, 2 * H, 2 * W, C)
